# baseline (speedup 1.0000x reference)
import functools

import jax
import jax.numpy as jnp
from jax import lax
from jax.experimental import pallas as pl
from jax.experimental.pallas import tpu as pltpu
from jax.experimental.pallas import tpu_sc as plsc

N_ROI_PER_GT = 16
FG_THRESH = 0.5
STRIDE = 16.0
POOL = 7
NPTS = POOL * POOL
NGATHER = 56
IDX_COLS = 64
G = 32
NP_PAD = 2048
R = G * N_ROI_PER_GT
HF = WF = 64
C = 768
D = 256
K_CLS = 5532
K_PAD = 5632
KB = 512

_LIN = [(j + 0.5) / POOL for j in range(POOL)]


def _match_body(gtb_ref, propsT_ref, lin_ref, val_ref):
    gx1 = gtb_ref[:, 0:1]
    gy1 = gtb_ref[:, 1:2]
    gx2 = gtb_ref[:, 2:3]
    gy2 = gtb_ref[:, 3:4]
    px1 = propsT_ref[0:1, :]
    py1 = propsT_ref[1:2, :]
    px2 = propsT_ref[2:3, :]
    py2 = propsT_ref[3:4, :]
    area_g = (gx2 - gx1) * (gy2 - gy1)
    area_p = (px2 - px1) * (py2 - py1)
    w = jnp.clip(jnp.minimum(gx2, px2) - jnp.maximum(gx1, px1), 0.0)
    h = jnp.clip(jnp.minimum(gy2, py2) - jnp.maximum(gy1, py1), 0.0)
    inter = w * h
    iou = inter / jnp.maximum(area_g + area_p - inter, 1e-9)

    mx = jnp.max(iou, axis=0, keepdims=True)
    matched = (iou == mx) & (iou >= FG_THRESH)
    thr = jnp.where(matched, iou, 0.0)

    colid = lax.broadcasted_iota(jnp.int32, (G, NP_PAD), 1)
    col64 = lax.broadcasted_iota(jnp.int32, (G, IDX_COLS), 1)
    kcol = lax.broadcasted_iota(jnp.int32, (G, 128), 1)
    a_of = col64 // POOL
    b_of = col64 % POOL
    valmat = jnp.zeros((G, 128), jnp.float32)

    for k in range(N_ROI_PER_GT):
        rowmax = jnp.max(thr, axis=1, keepdims=True)
        ismax = (thr == rowmax) & (rowmax > 0)
        arg = jnp.min(jnp.where(ismax, colid, jnp.int32(1 << 30)),
                      axis=1, keepdims=True)
        picked = colid == arg
        thr = jnp.where(picked, 0.0, thr)
        valmat = valmat + jnp.where(kcol == k, rowmax, 0.0)
        pickedf = picked.astype(jnp.float32)
        bx1 = jnp.sum(pickedf * px1, axis=1, keepdims=True)
        by1 = jnp.sum(pickedf * py1, axis=1, keepdims=True)
        bx2 = jnp.sum(pickedf * px2, axis=1, keepdims=True)
        by2 = jnp.sum(pickedf * py2, axis=1, keepdims=True)
        ysel = jnp.zeros((G, IDX_COLS), jnp.int32)
        xsel = jnp.zeros((G, IDX_COLS), jnp.int32)
        for j in range(POOL):
            xs = bx1 + (bx2 - bx1) * _LIN[j]
            ys = by1 + (by2 - by1) * _LIN[j]
            xi = jnp.clip(jnp.floor(xs / STRIDE).astype(jnp.int32), 0, WF - 1)
            yi = jnp.clip(jnp.floor(ys / STRIDE).astype(jnp.int32), 0, HF - 1)
            ysel = jnp.where(a_of == j, yi, ysel)
            xsel = jnp.where(b_of == j, xi, xsel)
        linmat = jnp.where(col64 < NPTS, ysel * WF + xsel, 0)
        lin_ref[:, k, :] = linmat
    val_ref[:, :] = valmat


_match_call = pl.pallas_call(
    _match_body,
    out_shape=(jax.ShapeDtypeStruct((G, N_ROI_PER_GT, IDX_COLS), jnp.int32),
               jax.ShapeDtypeStruct((G, 128), jnp.float32)),
)


_NC, _NS, _L = 2, 16, 16
_NW = _NC * _NS
_RPW = R // _NW

def _pool_body(feat_hbm, idx_hbm, out_hbm, idx_v, rows_v, out_v, sem):
    wid = lax.axis_index("s") * _NC + lax.axis_index("c")
    base = wid * _RPW
    pltpu.sync_copy(idx_hbm.at[pl.ds(base, _RPW)], idx_v)
    for i in range(_RPW):
        pltpu.async_copy(
            feat_hbm.at[idx_v.at[i, pl.ds(0, NGATHER)]], rows_v, sem).wait()

        def _dbody(d, carry):
            s = pl.ds(d * _L, _L)
            acc = rows_v[0, s]
            for jj in range(1, NPTS):
                acc = acc + rows_v[jj, s]
            out_v[i, s] = acc * (1.0 / NPTS)
            return carry

        lax.fori_loop(0, C // _L, _dbody, 0)
    pltpu.sync_copy(out_v, out_hbm.at[pl.ds(base, _RPW)])


@functools.cache
def _pool_call():
    mesh = plsc.VectorSubcoreMesh(
        core_axis_name="c", subcore_axis_name="s", num_cores=_NC)
    return pl.kernel(
        _pool_body,
        mesh=mesh,
        out_type=jax.ShapeDtypeStruct((R, C), jnp.float32),
        scratch_types=[
            pltpu.VMEM((_RPW, IDX_COLS), jnp.int32),
            pltpu.VMEM((NGATHER, C), jnp.float32),
            pltpu.VMEM((_RPW, C), jnp.float32),
            pltpu.SemaphoreType.DMA,
        ],
    )


def _head_body(pooled_ref, we_ref, val_ref, wc_ref, out_ref, emb_ref):
    j = pl.program_id(0)

    @pl.when(j == 0)
    def _():
        emb = jnp.dot(pooled_ref[...], we_ref[...],
                      preferred_element_type=jnp.float32)
        nrm = jnp.sqrt(jnp.sum(emb * emb, axis=1, keepdims=True))
        validf = (val_ref[...] > 0).astype(jnp.float32)
        emb_ref[...] = emb / jnp.maximum(nrm, 1e-12) * validf

    out_ref[...] = jnp.dot(emb_ref[...], wc_ref[...],
                           preferred_element_type=jnp.float32)


_head_call = pl.pallas_call(
    _head_body,
    grid=(K_PAD // KB,),
    in_specs=[
        pl.BlockSpec((R, C), lambda j: (0, 0)),
        pl.BlockSpec((C, D), lambda j: (0, 0)),
        pl.BlockSpec((R, 1), lambda j: (0, 0)),
        pl.BlockSpec((D, KB), lambda j: (0, j)),
    ],
    out_specs=pl.BlockSpec((R, KB), lambda j: (0, j)),
    out_shape=jax.ShapeDtypeStruct((R, K_PAD), jnp.float32),
    scratch_shapes=[pltpu.VMEM((R, D), jnp.float32)],
)


def kernel(features, proposals, gt_boxes, gt_pids, W_extract, W_cls):
    n = proposals.shape[0]
    propsT = jnp.zeros((8, NP_PAD), jnp.float32)
    propsT = propsT.at[0:4, 0:n].set(proposals.T)
    propsT = propsT.at[0:4, n:n + G].set(gt_boxes.T)
    gtb = jnp.zeros((G, 128), jnp.float32).at[:, 0:4].set(gt_boxes)

    lin_out, val_out = _match_call(gtb, propsT)
    lin_idx = lin_out.reshape(R, IDX_COLS)
    val512 = val_out[:, :N_ROI_PER_GT].reshape(R, 1)

    feat2 = features.reshape(C, HF * WF).T
    pooled = _pool_call()(feat2, lin_idx)

    wc_pad = jnp.pad(W_cls, ((0, 0), (0, K_PAD - K_CLS)))
    logits = _head_call(pooled, W_extract, val512, wc_pad)
    return logits[:, :K_CLS]


# speedup vs baseline: 10.0478x; 10.0478x over previous
import functools

import jax
import jax.numpy as jnp
from jax import lax
from jax.experimental import pallas as pl
from jax.experimental.pallas import tpu as pltpu
from jax.experimental.pallas import tpu_sc as plsc

N_ROI_PER_GT = 16
FG_THRESH = 0.5
STRIDE = 16.0
POOL = 7
NPTS = POOL * POOL
NGATHER = 56
IDX_COLS = 64
G = 32
NP_PAD = 2048
R = G * N_ROI_PER_GT
HF = WF = 64
C = 768
D = 256
K_CLS = 5532
K_PAD = 5632
KB = 512

_LIN = [(j + 0.5) / POOL for j in range(POOL)]


def _match_body(gtb_ref, propsT_ref, lin_ref, val_ref, flg_ref):
    gx1 = gtb_ref[:, 0:1]
    gy1 = gtb_ref[:, 1:2]
    gx2 = gtb_ref[:, 2:3]
    gy2 = gtb_ref[:, 3:4]
    px1 = propsT_ref[0:1, :]
    py1 = propsT_ref[1:2, :]
    px2 = propsT_ref[2:3, :]
    py2 = propsT_ref[3:4, :]
    area_g = (gx2 - gx1) * (gy2 - gy1)
    area_p = (px2 - px1) * (py2 - py1)
    w = jnp.clip(jnp.minimum(gx2, px2) - jnp.maximum(gx1, px1), 0.0)
    h = jnp.clip(jnp.minimum(gy2, py2) - jnp.maximum(gy1, py1), 0.0)
    inter = w * h
    iou = inter / jnp.maximum(area_g + area_p - inter, 1e-9)

    mx = jnp.max(iou, axis=0, keepdims=True)
    matched = (iou == mx) & (iou >= FG_THRESH)
    thr = jnp.where(matched, iou, 0.0)

    colid = lax.broadcasted_iota(jnp.int32, (G, NP_PAD), 1)
    col64 = lax.broadcasted_iota(jnp.int32, (G, IDX_COLS), 1)
    kcol = lax.broadcasted_iota(jnp.int32, (G, 128), 1)
    a_of = col64 // POOL
    b_of = col64 % POOL
    valmat = jnp.zeros((G, 128), jnp.float32)
    flgmat = jnp.zeros((G, 128), jnp.int32)

    for k in range(N_ROI_PER_GT):
        rowmax = jnp.max(thr, axis=1, keepdims=True)
        ismax = (thr == rowmax) & (rowmax > 0)
        arg = jnp.min(jnp.where(ismax, colid, jnp.int32(1 << 30)),
                      axis=1, keepdims=True)
        picked = colid == arg
        thr = jnp.where(picked, 0.0, thr)
        valmat = valmat + jnp.where(kcol == k, rowmax, 0.0)
        pickedf = picked.astype(jnp.float32)
        bx1 = jnp.sum(pickedf * px1, axis=1, keepdims=True)
        by1 = jnp.sum(pickedf * py1, axis=1, keepdims=True)
        bx2 = jnp.sum(pickedf * px2, axis=1, keepdims=True)
        by2 = jnp.sum(pickedf * py2, axis=1, keepdims=True)
        ysel = jnp.zeros((G, IDX_COLS), jnp.int32)
        xsel = jnp.zeros((G, IDX_COLS), jnp.int32)
        for j in range(POOL):
            xs = bx1 + (bx2 - bx1) * _LIN[j]
            ys = by1 + (by2 - by1) * _LIN[j]
            xi = jnp.clip(jnp.floor(xs / STRIDE).astype(jnp.int32), 0, WF - 1)
            yi = jnp.clip(jnp.floor(ys / STRIDE).astype(jnp.int32), 0, HF - 1)
            ysel = jnp.where(a_of == j, yi, ysel)
            xsel = jnp.where(b_of == j, xi, xsel)
        flgmat = flgmat + jnp.where(kcol == k,
                                    (rowmax > 0).astype(jnp.int32), 0)
        linmat = jnp.where(col64 < NPTS, ysel * WF + xsel, 0)
        lin_ref[:, k, :] = linmat
    val_ref[:, :] = valmat
    flg_ref[:, :] = flgmat


_match_call = pl.pallas_call(
    _match_body,
    out_shape=(jax.ShapeDtypeStruct((G, N_ROI_PER_GT, IDX_COLS), jnp.int32),
               jax.ShapeDtypeStruct((G, 128), jnp.float32),
               jax.ShapeDtypeStruct((G, 128), jnp.int32)),
)


_NC, _NS, _L = 2, 16, 16
_NW = _NC * _NS
_RPW = R // _NW

def _pool_body(feat_hbm, idx_hbm, flg_hbm, out_hbm, idx_v, flg_v, rows_v,
               out_v, sem):
    wid = lax.axis_index("s") * _NC + lax.axis_index("c")
    base = wid * _RPW
    pltpu.sync_copy(idx_hbm.at[pl.ds(base, _RPW)], idx_v)
    pltpu.sync_copy(flg_hbm.at[wid], flg_v)
    lane = lax.iota(jnp.int32, 16)
    flags = flg_v[...]
    for i in range(_RPW):
        fi = jnp.max(jnp.where(lane == i, flags, 0.0))

        @pl.when(fi > 0.0)
        def _():
            pltpu.async_copy(
                feat_hbm.at[idx_v.at[i, pl.ds(0, NGATHER)]], rows_v,
                sem).wait()

            def _dbody(d, carry):
                s = pl.ds(d * _L, _L)
                acc = rows_v[0, s]
                for jj in range(1, NPTS):
                    acc = acc + rows_v[jj, s]
                out_v[i, s] = acc * (1.0 / NPTS)
                return carry

            lax.fori_loop(0, C // _L, _dbody, 0)

        @pl.when(fi <= 0.0)
        def _():
            def _zbody(d, carry):
                out_v[i, pl.ds(d * _L, _L)] = jnp.zeros((_L,), jnp.float32)
                return carry

            lax.fori_loop(0, C // _L, _zbody, 0)
    pltpu.sync_copy(out_v, out_hbm.at[pl.ds(base, _RPW)])


@functools.cache
def _pool_call():
    mesh = plsc.VectorSubcoreMesh(
        core_axis_name="c", subcore_axis_name="s", num_cores=_NC)
    return pl.kernel(
        _pool_body,
        mesh=mesh,
        compiler_params=pltpu.CompilerParams(needs_layout_passes=False),
        out_type=jax.ShapeDtypeStruct((R, C), jnp.float32),
        scratch_types=[
            pltpu.VMEM((_RPW, IDX_COLS), jnp.int32),
            pltpu.VMEM((16,), jnp.float32),
            pltpu.VMEM((NGATHER, C), jnp.float32),
            pltpu.VMEM((_RPW, C), jnp.float32),
            pltpu.SemaphoreType.DMA,
        ],
    )


def _head_body(pooled_ref, we_ref, val_ref, wc_ref, out_ref, emb_ref):
    j = pl.program_id(0)

    @pl.when(j == 0)
    def _():
        emb = jnp.dot(pooled_ref[...], we_ref[...],
                      preferred_element_type=jnp.float32)
        nrm = jnp.sqrt(jnp.sum(emb * emb, axis=1, keepdims=True))
        validf = (val_ref[...] > 0).astype(jnp.float32)
        emb_ref[...] = emb / jnp.maximum(nrm, 1e-12) * validf

    out_ref[...] = jnp.dot(emb_ref[...], wc_ref[...],
                           preferred_element_type=jnp.float32)


_head_call = pl.pallas_call(
    _head_body,
    grid=(K_PAD // KB,),
    in_specs=[
        pl.BlockSpec((R, C), lambda j: (0, 0)),
        pl.BlockSpec((C, D), lambda j: (0, 0)),
        pl.BlockSpec((R, 1), lambda j: (0, 0)),
        pl.BlockSpec((D, KB), lambda j: (0, j)),
    ],
    out_specs=pl.BlockSpec((R, KB), lambda j: (0, j)),
    out_shape=jax.ShapeDtypeStruct((R, K_PAD), jnp.float32),
    scratch_shapes=[pltpu.VMEM((R, D), jnp.float32)],
)


def kernel(features, proposals, gt_boxes, gt_pids, W_extract, W_cls):
    n = proposals.shape[0]
    propsT = jnp.zeros((8, NP_PAD), jnp.float32)
    propsT = propsT.at[0:4, 0:n].set(proposals.T)
    propsT = propsT.at[0:4, n:n + G].set(gt_boxes.T)
    gtb = jnp.zeros((G, 128), jnp.float32).at[:, 0:4].set(gt_boxes)

    lin_out, val_out, flg_out = _match_call(gtb, propsT)
    lin_idx = lin_out.reshape(R, IDX_COLS)
    val512 = val_out[:, :N_ROI_PER_GT].reshape(R, 1)
    flg32 = val_out[:, :N_ROI_PER_GT]

    feat2 = features.reshape(C, HF * WF).T
    pooled = _pool_call()(feat2, lin_idx, flg32)

    wc_pad = jnp.pad(W_cls, ((0, 0), (0, K_PAD - K_CLS)))
    logits = _head_call(pooled, W_extract, val512, wc_pad)
    return logits[:, :K_CLS]
